# Initial kernel scaffold; baseline (speedup 1.0000x reference)
import jax
import jax.numpy as jnp
from jax.experimental import pallas as pl


def kernel(input, target):
    raise NotImplementedError("write your pallas kernel here")


# speedup vs baseline: 1.3758x; 1.3758x over previous
import jax
import jax.numpy as jnp
from jax.experimental import pallas as pl
from jax.experimental.pallas import tpu as pltpu

B, H, W = 128, 480, 640
N = H * W
RB = 8
CH = 96
GROUPS = B // RB
KSTEPS = H // CH


def _rowsum(v):
    return jnp.sum(jnp.sum(v, axis=1), axis=1, keepdims=True)


def _stats_kernel(x_ref, t_ref, s1_ref, s2_ref, s3_ref, s4_ref):
    k = pl.program_id(1)
    x = x_ref[...]
    t = t_ref[...]
    nz = t != 0.0
    s1 = _rowsum(x)
    s2 = _rowsum(x * x)
    s3 = _rowsum(jnp.where(nz, x, 0.0))
    s4 = _rowsum(jnp.where(nz, 1.0, 0.0))

    @pl.when(k == 0)
    def _():
        s1_ref[...] = jnp.zeros_like(s1_ref)
        s2_ref[...] = jnp.zeros_like(s2_ref)
        s3_ref[...] = jnp.zeros_like(s3_ref)
        s4_ref[...] = jnp.zeros_like(s4_ref)

    s1_ref[...] += jnp.broadcast_to(s1, (RB, 128))
    s2_ref[...] += jnp.broadcast_to(s2, (RB, 128))
    s3_ref[...] += jnp.broadcast_to(s3, (RB, 128))
    s4_ref[...] += jnp.broadcast_to(s4, (RB, 128))


def _combine_kernel(s1_ref, s2_ref, s3_ref, s4_ref, out_ref):
    s1 = s1_ref[...]
    s2 = s2_ref[...]
    s3 = s3_ref[...]
    s4 = s4_ref[...]
    n = jnp.float32(N)
    mean = s1 / n
    var = (s2 - s1 * mean) / jnp.float32(N - 1)
    inv_std = jax.lax.rsqrt(var)
    nss = (s3 - mean * s4) * inv_std * jnp.float32(1.0 / N)
    tot = jnp.sum(nss, axis=0, keepdims=True) * jnp.float32(1.0 / B)
    out_ref[...] = jnp.broadcast_to(tot, (8, 128))


def kernel(input, target):
    stat_shape = jax.ShapeDtypeStruct((B, 128), jnp.float32)
    in_spec = pl.BlockSpec((RB, CH, W), lambda g, k: (g, k, 0))
    out_spec = pl.BlockSpec((RB, 128), lambda g, k: (g, 0))
    s1, s2, s3, s4 = pl.pallas_call(
        _stats_kernel,
        grid=(GROUPS, KSTEPS),
        in_specs=[in_spec, in_spec],
        out_specs=[out_spec, out_spec, out_spec, out_spec],
        out_shape=[stat_shape, stat_shape, stat_shape, stat_shape],
        compiler_params=pltpu.CompilerParams(
            dimension_semantics=("parallel", "arbitrary"),
        ),
    )(input, target)

    out = pl.pallas_call(
        _combine_kernel,
        out_shape=jax.ShapeDtypeStruct((8, 128), jnp.float32),
    )(s1, s2, s3, s4)
    return out[0, 0]
